# Initial kernel scaffold; baseline (speedup 1.0000x reference)
import jax
import jax.numpy as jnp
from jax.experimental import pallas as pl


def kernel(x, q, E0, E1, E2, E3, T0, T1, T2, T3):
    raise NotImplementedError("write your pallas kernel here")


# speedup vs baseline: 9.2455x; 9.2455x over previous
import jax
import jax.numpy as jnp
from jax import lax
from jax.experimental import pallas as pl
from jax.experimental.pallas import tpu as pltpu
from jax.experimental.pallas import tpu_sc as plsc

V = 100000
D = 64
S = 20
M = 50
B = 1024

NT = 32
BAGS = B * M
BAGS_PER_TILE = BAGS // NT
CHUNK_BAGS = 32
CHUNKS = BAGS_PER_TILE // CHUNK_BAGS
IDX_ROWS = CHUNK_BAGS * S // 128
Q_BAGS_PER_TILE = B // NT

_A = [1.0 - (s + 1) / S for s in range(S)]
_Bw = [1.0 - 2.0 * (s + 1) / S for s in range(S)]


CHUNK_IDX = CHUNK_BAGS * S


def _sc_embed_body(x1d, q1d, e0, e1, e2, e3, g_out, u_out, idx_v, rows_v, out_v, sem):
    cid = lax.axis_index("c")
    sid = lax.axis_index("s")
    wid = sid * 2 + cid
    tables = [e0, e1, e2, e3]

    ii = lax.broadcasted_iota(jnp.int32, (16,), 0).astype(jnp.float32)
    cvecs = [-(ii + float(1 + 16 * d4)) * (1.0 / D) for d4 in range(4)]

    def bag_compute(j, tslot):
        base = j * S
        acc_a = [None] * 4
        acc_b = [None] * 4
        for s in range(S):
            for d4 in range(4):
                r = rows_v[base + s, pl.ds(d4 * 16, 16)]
                if s == 0:
                    acc_a[d4] = _A[0] * r
                    acc_b[d4] = _Bw[0] * r
                else:
                    acc_a[d4] = acc_a[d4] + _A[s] * r
                    acc_b[d4] = acc_b[d4] + _Bw[s] * r
        for d4 in range(4):
            out_v[tslot, j, pl.ds(d4 * 16, 16)] = acc_a[d4] + cvecs[d4] * acc_b[d4]

    def make_bag_body(tslot):
        def body(j, carry):
            bag_compute(j, tslot)
            return carry
        return body

    def gather_and_reduce(table, tslot, nbags):
        cps = [
            pltpu.async_copy(
                table.at[idx_v.at[pl.ds(k * 128, 128)]],
                rows_v.at[pl.ds(k * 128, 128)],
                sem,
            )
            for k in range(IDX_ROWS)
        ]
        for c in cps:
            c.wait()
        lax.fori_loop(0, nbags, make_bag_body(tslot), 0)

    def chunk_body(ch, carry):
        bag_base = wid * BAGS_PER_TILE + ch * CHUNK_BAGS
        pltpu.sync_copy(x1d.at[pl.ds(bag_base * S, CHUNK_IDX)], idx_v)
        for t in range(4):
            gather_and_reduce(tables[t], t, CHUNK_BAGS)
            pltpu.sync_copy(
                out_v.at[t], g_out.at[pl.ds(t * BAGS + bag_base, CHUNK_BAGS)]
            )
        return carry

    lax.fori_loop(0, CHUNKS, chunk_body, 0)

    pltpu.sync_copy(q1d.at[pl.ds(wid * CHUNK_IDX, CHUNK_IDX)], idx_v)
    gather_and_reduce(e0, 0, Q_BAGS_PER_TILE)
    pltpu.sync_copy(out_v.at[0], u_out.at[pl.ds(wid * Q_BAGS_PER_TILE, Q_BAGS_PER_TILE)])


def _sc_embed(x1d, q1d, e0, e1, e2, e3):
    mesh = plsc.VectorSubcoreMesh(core_axis_name="c", subcore_axis_name="s")
    return pl.kernel(
        _sc_embed_body,
        out_type=(
            jax.ShapeDtypeStruct((4 * BAGS, D), jnp.float32),
            jax.ShapeDtypeStruct((B, D), jnp.float32),
        ),
        mesh=mesh,
        scratch_types=(
            pltpu.VMEM((CHUNK_IDX,), jnp.int32),
            pltpu.VMEM((CHUNK_BAGS * S, D), jnp.float32),
            pltpu.VMEM((4, CHUNK_BAGS, D), jnp.float32),
            pltpu.SemaphoreType.DMA,
        ),
        compiler_params=pltpu.CompilerParams(use_tc_tiling_on_sc=False),
    )(x1d, q1d, e0, e1, e2, e3)


BT = 128


def _hops_body(g_ref, u0_ref, t_ref, w_ref):
    u = u0_ref[...]
    o = None
    for i in range(3):
        m = g_ref[i] + t_ref[i][None, :, :]
        c = g_ref[i + 1] + t_ref[i + 1][None, :, :]
        scores = jnp.sum(m * u[:, None, :], axis=2)
        smax = jnp.max(scores, axis=1, keepdims=True)
        e = jnp.exp(scores - smax)
        p = e / jnp.sum(e, axis=1, keepdims=True)
        o = jnp.sum(p[:, :, None] * c, axis=1)
        u = o + u
    w_ref[...] = o + u


def _hops(g4, u0, tst):
    return pl.pallas_call(
        _hops_body,
        grid=(B // BT,),
        in_specs=[
            pl.BlockSpec((4, BT, M, D), lambda i: (0, i, 0, 0)),
            pl.BlockSpec((BT, D), lambda i: (i, 0)),
            pl.BlockSpec((4, M, D), lambda i: (0, 0, 0)),
        ],
        out_specs=pl.BlockSpec((BT, D), lambda i: (i, 0)),
        out_shape=jax.ShapeDtypeStruct((B, D), jnp.float32),
    )(g4, u0, tst)


VT = 2048
NV = (V + VT - 1) // VT


def _mm_body(w_ref, e3_ref, o_ref):
    o_ref[...] = lax.dot_general(
        w_ref[...], e3_ref[...],
        (((1,), (1,)), ((), ())),
        preferred_element_type=jnp.float32,
    )


def _mm(w, e3):
    return pl.pallas_call(
        _mm_body,
        grid=(NV,),
        in_specs=[
            pl.BlockSpec((B, D), lambda i: (0, 0)),
            pl.BlockSpec((VT, D), lambda i: (i, 0)),
        ],
        out_specs=pl.BlockSpec((B, VT), lambda i: (0, i)),
        out_shape=jax.ShapeDtypeStruct((B, V), jnp.float32),
    )(w, e3)


def kernel(x, q, E0, E1, E2, E3, T0, T1, T2, T3):
    x1d = x.astype(jnp.int32).reshape(B * M * S)
    q1d = q.astype(jnp.int32).reshape(B * S)
    g_flat, u0 = _sc_embed(x1d, q1d, E0, E1, E2, E3)
    g4 = g_flat.reshape(4, B, M, D)
    tst = jnp.stack([T0, T1, T2, T3])
    w = _hops(g4, u0, tst)
    return _mm(w, E3)


# speedup vs baseline: 12.4490x; 1.3465x over previous
import jax
import jax.numpy as jnp
from jax import lax
from jax.experimental import pallas as pl
from jax.experimental.pallas import tpu as pltpu
from jax.experimental.pallas import tpu_sc as plsc

V = 100000
D = 64
S = 20
M = 50
B = 1024

NT = 32
BAGS = B * M
BAGS_PER_TILE = BAGS // NT
CHUNK_BAGS = 32
CHUNKS = BAGS_PER_TILE // CHUNK_BAGS
IDX_ROWS = CHUNK_BAGS * S // 128
Q_BAGS_PER_TILE = B // NT

_A = [1.0 - (s + 1) / S for s in range(S)]
_Bw = [1.0 - 2.0 * (s + 1) / S for s in range(S)]


CHUNK_IDX = CHUNK_BAGS * S
PAIRS = CHUNKS // 2


def _sc_embed_body(x1d, q1d, e0, e1, e2, e3, g_out, u_out,
                   idx_v, rows_v, out_v, gsem0, gsem1, isem, osem0, osem1):
    cid = lax.axis_index("c")
    sid = lax.axis_index("s")
    wid = sid * 2 + cid
    tables = [e0, e1, e2, e3]
    gsems = [gsem0, gsem1]
    osems = [osem0, osem1]

    ii = lax.broadcasted_iota(jnp.int32, (16,), 0).astype(jnp.float32)
    cvecs = [-(ii + float(1 + 16 * d4)) * (1.0 / D) for d4 in range(4)]

    x_base = wid * BAGS_PER_TILE * S
    bag_base0 = wid * BAGS_PER_TILE

    def fire_gather(table, ip, rp):
        for k in range(IDX_ROWS):
            pltpu.async_copy(
                table.at[idx_v.at[ip, pl.ds(k * 128, 128)]],
                rows_v.at[rp, pl.ds(k * 128, 128)],
                gsems[rp],
            )

    def wait_gather(rp):
        for k in range(IDX_ROWS):
            pltpu.make_async_copy(
                e0.at[idx_v.at[0, pl.ds(k * 128, 128)]],
                rows_v.at[rp, pl.ds(k * 128, 128)],
                gsems[rp],
            ).wait()

    def compute_chunk(rp, op, tslot, nbags):
        def body(j, carry):
            base = j * S
            acc_a = [None] * 4
            acc_b = [None] * 4
            for s in range(S):
                for d4 in range(4):
                    r = rows_v[rp, base + s, pl.ds(d4 * 16, 16)]
                    if s == 0:
                        acc_a[d4] = _A[0] * r
                        acc_b[d4] = _Bw[0] * r
                    else:
                        acc_a[d4] = acc_a[d4] + _A[s] * r
                        acc_b[d4] = acc_b[d4] + _Bw[s] * r
            for d4 in range(4):
                out_v[op, tslot, j, pl.ds(d4 * 16, 16)] = (
                    acc_a[d4] + cvecs[d4] * acc_b[d4]
                )
            return carry
        lax.fori_loop(0, nbags, body, 0)

    def fire_out_stores(op, ch):
        for t in range(4):
            pltpu.async_copy(
                out_v.at[op, t],
                g_out.at[pl.ds(t * BAGS + bag_base0 + ch * CHUNK_BAGS, CHUNK_BAGS)],
                osems[op],
            )

    def drain_out_stores(op):
        for t in range(4):
            pltpu.make_async_copy(
                out_v.at[op, t],
                g_out.at[pl.ds(t * BAGS, CHUNK_BAGS)],
                osems[op],
            ).wait()

    def run_chunk(i, ch, op, idx_prefetch):
        @pl.when(i >= 1)
        def _():
            drain_out_stores(op)
        icp = None
        for t in range(4):
            wait_gather(t % 2)
            if t == 0:
                icp = idx_prefetch()
            if t < 3:
                fire_gather(tables[t + 1], op, (t + 1) % 2)
            else:
                icp.wait()
                fire_gather(e0, 1 - op, 0)
            compute_chunk(t % 2, op, t, CHUNK_BAGS)
        fire_out_stores(op, ch)

    pltpu.sync_copy(x1d.at[pl.ds(x_base, CHUNK_IDX)], idx_v.at[0])
    fire_gather(e0, 0, 0)

    def pair_body(i, carry):
        ch_a = 2 * i
        ch_b = 2 * i + 1

        def prefetch_b():
            return pltpu.async_copy(
                x1d.at[pl.ds(x_base + (ch_b) * CHUNK_IDX, CHUNK_IDX)],
                idx_v.at[1], isem)

        def prefetch_a2():
            @pl.when(i < PAIRS - 1)
            def _():
                pltpu.async_copy(
                    x1d.at[pl.ds(x_base + (ch_b + 1) * CHUNK_IDX, CHUNK_IDX)],
                    idx_v.at[0], isem)
            @pl.when(i == PAIRS - 1)
            def _():
                pltpu.async_copy(
                    q1d.at[pl.ds(wid * CHUNK_IDX, CHUNK_IDX)],
                    idx_v.at[0], isem)
            return pltpu.make_async_copy(
                x1d.at[pl.ds(0, CHUNK_IDX)], idx_v.at[0], isem)

        run_chunk(i, ch_a, 0, prefetch_b)
        run_chunk(i, ch_b, 1, prefetch_a2)
        return carry

    lax.fori_loop(0, PAIRS, pair_body, 0)

    wait_gather(0)
    drain_out_stores(0)
    compute_chunk(0, 0, 0, Q_BAGS_PER_TILE)
    drain_out_stores(1)
    pltpu.sync_copy(out_v.at[0, 0],
                    u_out.at[pl.ds(wid * Q_BAGS_PER_TILE, Q_BAGS_PER_TILE)])


def _sc_embed(x1d, q1d, e0, e1, e2, e3):
    mesh = plsc.VectorSubcoreMesh(core_axis_name="c", subcore_axis_name="s")
    return pl.kernel(
        _sc_embed_body,
        out_type=(
            jax.ShapeDtypeStruct((4 * BAGS, D), jnp.float32),
            jax.ShapeDtypeStruct((B, D), jnp.float32),
        ),
        mesh=mesh,
        scratch_types=(
            pltpu.VMEM((2, CHUNK_IDX), jnp.int32),
            pltpu.VMEM((2, CHUNK_BAGS * S, D), jnp.float32),
            pltpu.VMEM((2, 4, CHUNK_BAGS, D), jnp.float32),
            pltpu.SemaphoreType.DMA,
            pltpu.SemaphoreType.DMA,
            pltpu.SemaphoreType.DMA,
            pltpu.SemaphoreType.DMA,
            pltpu.SemaphoreType.DMA,
        ),
        compiler_params=pltpu.CompilerParams(use_tc_tiling_on_sc=False),
    )(x1d, q1d, e0, e1, e2, e3)


BT = 128


def _hops_body(g_ref, u0_ref, t_ref, w_ref):
    u = u0_ref[...]
    o = None
    for i in range(3):
        m = g_ref[i] + t_ref[i][None, :, :]
        c = g_ref[i + 1] + t_ref[i + 1][None, :, :]
        scores = jnp.sum(m * u[:, None, :], axis=2)
        smax = jnp.max(scores, axis=1, keepdims=True)
        e = jnp.exp(scores - smax)
        p = e / jnp.sum(e, axis=1, keepdims=True)
        o = jnp.sum(p[:, :, None] * c, axis=1)
        u = o + u
    w_ref[...] = o + u


def _hops(g4, u0, tst):
    return pl.pallas_call(
        _hops_body,
        grid=(B // BT,),
        in_specs=[
            pl.BlockSpec((4, BT, M, D), lambda i: (0, i, 0, 0)),
            pl.BlockSpec((BT, D), lambda i: (i, 0)),
            pl.BlockSpec((4, M, D), lambda i: (0, 0, 0)),
        ],
        out_specs=pl.BlockSpec((BT, D), lambda i: (i, 0)),
        out_shape=jax.ShapeDtypeStruct((B, D), jnp.float32),
    )(g4, u0, tst)


VT = 2048
NV = (V + VT - 1) // VT


def _mm_body(w_ref, e3_ref, o_ref):
    o_ref[...] = lax.dot_general(
        w_ref[...], e3_ref[...],
        (((1,), (1,)), ((), ())),
        preferred_element_type=jnp.float32,
    )


def _mm(w, e3):
    return pl.pallas_call(
        _mm_body,
        grid=(NV,),
        in_specs=[
            pl.BlockSpec((B, D), lambda i: (0, 0)),
            pl.BlockSpec((VT, D), lambda i: (i, 0)),
        ],
        out_specs=pl.BlockSpec((B, VT), lambda i: (0, i)),
        out_shape=jax.ShapeDtypeStruct((B, V), jnp.float32),
    )(w, e3)


def kernel(x, q, E0, E1, E2, E3, T0, T1, T2, T3):
    x1d = x.astype(jnp.int32).reshape(B * M * S)
    q1d = q.astype(jnp.int32).reshape(B * S)
    g_flat, u0 = _sc_embed(x1d, q1d, E0, E1, E2, E3)
    g4 = g_flat.reshape(4, B, M, D)
    tst = jnp.stack([T0, T1, T2, T3])
    w = _hops(g4, u0, tst)
    return _mm(w, E3)


# speedup vs baseline: 18.7359x; 1.5050x over previous
import jax
import jax.numpy as jnp
from jax import lax
from jax.experimental import pallas as pl
from jax.experimental.pallas import tpu as pltpu
from jax.experimental.pallas import tpu_sc as plsc

V = 100000
D = 64
S = 20
M = 50
B = 1024

NT = 32
BAGS = B * M
BAGS_PER_TILE = BAGS // NT
CHUNK_BAGS = 32
CHUNKS = BAGS_PER_TILE // CHUNK_BAGS
IDX_ROWS = CHUNK_BAGS * S // 128
CHUNK_IDX = CHUNK_BAGS * S
PAIRS = CHUNKS // 2
Q_BAGS_PER_TILE = B // NT

_A = [1.0 - (s + 1) / S for s in range(S)]
_Bw = [1.0 - 2.0 * (s + 1) / S for s in range(S)]


def _make_sc_body(with_query):
    def body(*args):
        if with_query:
            (x1d, q1d, table, g_out, u_out,
             idx_v, rows_v, out_v, gsem0, gsem1, isem0, isem1, osem0, osem1) = args
        else:
            (x1d, table, g_out,
             idx_v, rows_v, out_v, gsem0, gsem1, isem0, isem1, osem0, osem1) = args
        cid = lax.axis_index("c")
        sid = lax.axis_index("s")
        wid = sid * 2 + cid
        gsems = [gsem0, gsem1]
        isems = [isem0, isem1]
        osems = [osem0, osem1]

        ii = lax.broadcasted_iota(jnp.int32, (16,), 0).astype(jnp.float32)
        cvecs = [-(ii + float(1 + 16 * d4)) * (1.0 / D) for d4 in range(4)]

        x_base = wid * BAGS_PER_TILE * S
        bag_base0 = wid * BAGS_PER_TILE

        def fire_gather(ip, rp):
            for k in range(IDX_ROWS):
                pltpu.async_copy(
                    table.at[idx_v.at[ip, pl.ds(k * 128, 128)]],
                    rows_v.at[rp, pl.ds(k * 128, 128)],
                    gsems[rp],
                )

        def wait_gather(rp):
            for k in range(IDX_ROWS):
                pltpu.make_async_copy(
                    table.at[idx_v.at[0, pl.ds(k * 128, 128)]],
                    rows_v.at[rp, pl.ds(k * 128, 128)],
                    gsems[rp],
                ).wait()

        def compute_chunk(rp, op, nbags):
            def bag(j, carry):
                base = j * S
                acc_a = [None] * 4
                acc_b = [None] * 4
                for s in range(S):
                    for d4 in range(4):
                        r = rows_v[rp, base + s, pl.ds(d4 * 16, 16)]
                        if s == 0:
                            acc_a[d4] = _A[0] * r
                            acc_b[d4] = _Bw[0] * r
                        else:
                            acc_a[d4] = acc_a[d4] + _A[s] * r
                            acc_b[d4] = acc_b[d4] + _Bw[s] * r
                for d4 in range(4):
                    out_v[op, j, pl.ds(d4 * 16, 16)] = (
                        acc_a[d4] + cvecs[d4] * acc_b[d4]
                    )
                return carry
            lax.fori_loop(0, nbags, bag, 0)

        def fire_store(op, ch):
            pltpu.async_copy(
                out_v.at[op],
                g_out.at[pl.ds(bag_base0 + ch * CHUNK_BAGS, CHUNK_BAGS)],
                osems[op],
            )

        def drain_store(op):
            pltpu.make_async_copy(
                out_v.at[op],
                g_out.at[pl.ds(bag_base0, CHUNK_BAGS)],
                osems[op],
            ).wait()

        def fire_idx_load(ch_next, p):
            pltpu.async_copy(
                x1d.at[pl.ds(x_base + ch_next * CHUNK_IDX, CHUNK_IDX)],
                idx_v.at[p], isems[p])

        def run_chunk(i, ch, p):
            wait_gather(p)
            if with_query:
                @pl.when(i < PAIRS - 1)
                def _():
                    fire_idx_load(ch + 2, p)
                if p == 0:
                    @pl.when(i == PAIRS - 1)
                    def _():
                        pltpu.async_copy(
                            q1d.at[pl.ds(wid * CHUNK_IDX, CHUNK_IDX)],
                            idx_v.at[0], isems[0])
            else:
                @pl.when(i + p < PAIRS - (1 - p))
                def _():
                    fire_idx_load(ch + 2, p)
            if with_query or p == 0:
                pltpu.make_async_copy(
                    x1d.at[pl.ds(0, CHUNK_IDX)], idx_v.at[1 - p], isems[1 - p]
                ).wait()
                fire_gather(1 - p, 1 - p)
            else:
                @pl.when(i < PAIRS - 1)
                def _():
                    pltpu.make_async_copy(
                        x1d.at[pl.ds(0, CHUNK_IDX)], idx_v.at[0], isems[0]
                    ).wait()
                    fire_gather(0, 0)
            @pl.when(i >= 1)
            def _():
                drain_store(p)
            compute_chunk(p, p, CHUNK_BAGS)
            fire_store(p, ch)

        pltpu.sync_copy(x1d.at[pl.ds(x_base, CHUNK_IDX)], idx_v.at[0])
        fire_gather(0, 0)
        fire_idx_load(1, 1)

        def pair_body(i, carry):
            run_chunk(i, 2 * i, 0)
            run_chunk(i, 2 * i + 1, 1)
            return carry
        lax.fori_loop(0, PAIRS, pair_body, 0)

        if with_query:
            wait_gather(0)
            drain_store(0)
            compute_chunk(0, 0, Q_BAGS_PER_TILE)
            drain_store(1)
            pltpu.sync_copy(
                out_v.at[0],
                u_out.at[pl.ds(wid * Q_BAGS_PER_TILE, Q_BAGS_PER_TILE)])
        else:
            drain_store(0)
            drain_store(1)
    return body


_SC_SCRATCH = (
    pltpu.VMEM((2, CHUNK_IDX), jnp.int32),
    pltpu.VMEM((2, CHUNK_IDX, D), jnp.float32),
    pltpu.VMEM((2, CHUNK_BAGS, D), jnp.float32),
    pltpu.SemaphoreType.DMA,
    pltpu.SemaphoreType.DMA,
    pltpu.SemaphoreType.DMA,
    pltpu.SemaphoreType.DMA,
    pltpu.SemaphoreType.DMA,
    pltpu.SemaphoreType.DMA,
)


def _sc_embed_q(x1d, q1d, e0):
    mesh = plsc.VectorSubcoreMesh(core_axis_name="c", subcore_axis_name="s")
    return pl.kernel(
        _make_sc_body(True),
        out_type=(
            jax.ShapeDtypeStruct((BAGS, D), jnp.float32),
            jax.ShapeDtypeStruct((B, D), jnp.float32),
        ),
        mesh=mesh,
        scratch_types=_SC_SCRATCH,
        compiler_params=pltpu.CompilerParams(use_tc_tiling_on_sc=False),
        name="sc_embed_q",
    )(x1d, q1d, e0)


def _sc_embed(x1d, table):
    mesh = plsc.VectorSubcoreMesh(core_axis_name="c", subcore_axis_name="s")
    return pl.kernel(
        _make_sc_body(False),
        out_type=jax.ShapeDtypeStruct((BAGS, D), jnp.float32),
        mesh=mesh,
        scratch_types=_SC_SCRATCH,
        compiler_params=pltpu.CompilerParams(use_tc_tiling_on_sc=False),
        name="sc_embed",
    )(x1d, table)


BT = 128


def _hops_body(g0_ref, g1_ref, g2_ref, g3_ref, u0_ref, t_ref, w_ref):
    g_refs = [g0_ref, g1_ref, g2_ref, g3_ref]
    u = u0_ref[...]
    o = None
    for i in range(3):
        m = g_refs[i][...] + t_ref[i][None, :, :]
        c = g_refs[i + 1][...] + t_ref[i + 1][None, :, :]
        scores = jnp.sum(m * u[:, None, :], axis=2)
        smax = jnp.max(scores, axis=1, keepdims=True)
        e = jnp.exp(scores - smax)
        p = e / jnp.sum(e, axis=1, keepdims=True)
        o = jnp.sum(p[:, :, None] * c, axis=1)
        u = o + u
    w_ref[...] = o + u


def _hops(g4, u0, tst):
    gspec = pl.BlockSpec((BT, M, D), lambda i: (i, 0, 0))
    return pl.pallas_call(
        _hops_body,
        grid=(B // BT,),
        in_specs=[
            gspec, gspec, gspec, gspec,
            pl.BlockSpec((BT, D), lambda i: (i, 0)),
            pl.BlockSpec((4, M, D), lambda i: (0, 0, 0)),
        ],
        out_specs=pl.BlockSpec((BT, D), lambda i: (i, 0)),
        out_shape=jax.ShapeDtypeStruct((B, D), jnp.float32),
    )(*g4, u0, tst)


VT = 2048
NV = (V + VT - 1) // VT


def _mm_body(e3t_ref, w_ref, o_ref):
    o_ref[...] = lax.dot_general(
        e3t_ref[...], w_ref[...],
        (((0,), (1,)), ((), ())),
        preferred_element_type=jnp.float32,
    )


def _mm(w, e3t):
    return pl.pallas_call(
        _mm_body,
        grid=(NV,),
        in_specs=[
            pl.BlockSpec((D, VT), lambda i: (0, i)),
            pl.BlockSpec((B, D), lambda i: (0, 0)),
        ],
        out_specs=pl.BlockSpec((VT, B), lambda i: (i, 0)),
        out_shape=jax.ShapeDtypeStruct((V, B), jnp.float32),
    )(e3t, w)


def kernel(x, q, E0, E1, E2, E3, T0, T1, T2, T3):
    x1d = x.astype(jnp.int32).reshape(B * M * S)
    q1d = q.astype(jnp.int32).reshape(B * S)
    g0, u0 = _sc_embed_q(x1d, q1d, E0)
    g1 = _sc_embed(x1d, E1)
    g2 = _sc_embed(x1d, E2)
    g3 = _sc_embed(x1d, E3)
    g4 = [g.reshape(B, M, D) for g in (g0, g1, g2, g3)]
    tst = jnp.stack([T0, T1, T2, T3])
    w = _hops(g4, u0, tst)
    out_t = _mm(w, E3.T)
    return out_t.T


# speedup vs baseline: 18.7398x; 1.0002x over previous
import jax
import jax.numpy as jnp
from jax import lax
from jax.experimental import pallas as pl
from jax.experimental.pallas import tpu as pltpu
from jax.experimental.pallas import tpu_sc as plsc

V = 100000
D = 64
S = 20
M = 50
B = 1024

NT = 32
BAGS = B * M
BAGS_PER_TILE = BAGS // NT
CHUNK_BAGS = 32
CHUNKS = BAGS_PER_TILE // CHUNK_BAGS
IDX_ROWS = CHUNK_BAGS * S // 128
CHUNK_IDX = CHUNK_BAGS * S
PAIRS = CHUNKS // 2
Q_BAGS_PER_TILE = B // NT

_A = [1.0 - (s + 1) / S for s in range(S)]
_Bw = [1.0 - 2.0 * (s + 1) / S for s in range(S)]


def _make_sc_body(with_query):
    def body(*args):
        if with_query:
            (x1d, q1d, table, g_out, u_out,
             idx_v, rows_v, out_v, gsem0, gsem1, isem0, isem1, osem0, osem1) = args
        else:
            (x1d, table, g_out,
             idx_v, rows_v, out_v, gsem0, gsem1, isem0, isem1, osem0, osem1) = args
        cid = lax.axis_index("c")
        sid = lax.axis_index("s")
        wid = sid * 2 + cid
        gsems = [gsem0, gsem1]
        isems = [isem0, isem1]
        osems = [osem0, osem1]

        ii = lax.broadcasted_iota(jnp.int32, (16,), 0).astype(jnp.float32)
        cvecs = [-(ii + float(1 + 16 * d4)) * (1.0 / D) for d4 in range(4)]

        x_base = wid * BAGS_PER_TILE * S
        bag_base0 = wid * BAGS_PER_TILE

        def fire_gather(ip, rp):
            pltpu.async_copy(
                table.at[idx_v.at[ip]],
                rows_v.at[rp],
                gsems[rp],
            )

        def wait_gather(rp):
            pltpu.make_async_copy(
                table.at[idx_v.at[0]],
                rows_v.at[rp],
                gsems[rp],
            ).wait()

        def compute_chunk(rp, op, nbags):
            def bag(j, carry):
                base = j * S
                acc_a = [None] * 4
                acc_b = [None] * 4
                for s in range(S):
                    for d4 in range(4):
                        r = rows_v[rp, base + s, pl.ds(d4 * 16, 16)]
                        if s == 0:
                            acc_a[d4] = _A[0] * r
                            acc_b[d4] = _Bw[0] * r
                        else:
                            acc_a[d4] = acc_a[d4] + _A[s] * r
                            acc_b[d4] = acc_b[d4] + _Bw[s] * r
                for d4 in range(4):
                    out_v[op, j, pl.ds(d4 * 16, 16)] = (
                        acc_a[d4] + cvecs[d4] * acc_b[d4]
                    )
                return carry
            lax.fori_loop(0, nbags, bag, 0)

        def fire_store(op, ch):
            pltpu.async_copy(
                out_v.at[op],
                g_out.at[pl.ds(bag_base0 + ch * CHUNK_BAGS, CHUNK_BAGS)],
                osems[op],
            )

        def drain_store(op):
            pltpu.make_async_copy(
                out_v.at[op],
                g_out.at[pl.ds(bag_base0, CHUNK_BAGS)],
                osems[op],
            ).wait()

        def fire_idx_load(ch_next, p):
            pltpu.async_copy(
                x1d.at[pl.ds(x_base + ch_next * CHUNK_IDX, CHUNK_IDX)],
                idx_v.at[p], isems[p])

        def run_chunk(i, ch, p):
            wait_gather(p)
            if with_query:
                @pl.when(i < PAIRS - 1)
                def _():
                    fire_idx_load(ch + 2, p)
                if p == 0:
                    @pl.when(i == PAIRS - 1)
                    def _():
                        pltpu.async_copy(
                            q1d.at[pl.ds(wid * CHUNK_IDX, CHUNK_IDX)],
                            idx_v.at[0], isems[0])
            else:
                @pl.when(i + p < PAIRS - (1 - p))
                def _():
                    fire_idx_load(ch + 2, p)
            if with_query or p == 0:
                pltpu.make_async_copy(
                    x1d.at[pl.ds(0, CHUNK_IDX)], idx_v.at[1 - p], isems[1 - p]
                ).wait()
                fire_gather(1 - p, 1 - p)
            else:
                @pl.when(i < PAIRS - 1)
                def _():
                    pltpu.make_async_copy(
                        x1d.at[pl.ds(0, CHUNK_IDX)], idx_v.at[0], isems[0]
                    ).wait()
                    fire_gather(0, 0)
            @pl.when(i >= 1)
            def _():
                drain_store(p)
            compute_chunk(p, p, CHUNK_BAGS)
            fire_store(p, ch)

        pltpu.sync_copy(x1d.at[pl.ds(x_base, CHUNK_IDX)], idx_v.at[0])
        fire_gather(0, 0)
        fire_idx_load(1, 1)

        def pair_body(i, carry):
            run_chunk(i, 2 * i, 0)
            run_chunk(i, 2 * i + 1, 1)
            return carry
        lax.fori_loop(0, PAIRS, pair_body, 0)

        if with_query:
            wait_gather(0)
            drain_store(0)
            compute_chunk(0, 0, Q_BAGS_PER_TILE)
            drain_store(1)
            pltpu.sync_copy(
                out_v.at[0],
                u_out.at[pl.ds(wid * Q_BAGS_PER_TILE, Q_BAGS_PER_TILE)])
        else:
            drain_store(0)
            drain_store(1)
    return body


_SC_SCRATCH = (
    pltpu.VMEM((2, CHUNK_IDX), jnp.int32),
    pltpu.VMEM((2, CHUNK_IDX, D), jnp.float32),
    pltpu.VMEM((2, CHUNK_BAGS, D), jnp.float32),
    pltpu.SemaphoreType.DMA,
    pltpu.SemaphoreType.DMA,
    pltpu.SemaphoreType.DMA,
    pltpu.SemaphoreType.DMA,
    pltpu.SemaphoreType.DMA,
    pltpu.SemaphoreType.DMA,
)


def _sc_embed_q(x1d, q1d, e0):
    mesh = plsc.VectorSubcoreMesh(core_axis_name="c", subcore_axis_name="s")
    return pl.kernel(
        _make_sc_body(True),
        out_type=(
            jax.ShapeDtypeStruct((BAGS, D), jnp.float32),
            jax.ShapeDtypeStruct((B, D), jnp.float32),
        ),
        mesh=mesh,
        scratch_types=_SC_SCRATCH,
        compiler_params=pltpu.CompilerParams(use_tc_tiling_on_sc=False),
        name="sc_embed_q",
    )(x1d, q1d, e0)


def _sc_embed(x1d, table):
    mesh = plsc.VectorSubcoreMesh(core_axis_name="c", subcore_axis_name="s")
    return pl.kernel(
        _make_sc_body(False),
        out_type=jax.ShapeDtypeStruct((BAGS, D), jnp.float32),
        mesh=mesh,
        scratch_types=_SC_SCRATCH,
        compiler_params=pltpu.CompilerParams(use_tc_tiling_on_sc=False),
        name="sc_embed",
    )(x1d, table)


BT = 128


def _hops_body(g0_ref, g1_ref, g2_ref, g3_ref, u0_ref, t_ref, w_ref):
    g_refs = [g0_ref, g1_ref, g2_ref, g3_ref]
    u = u0_ref[...]
    o = None
    for i in range(3):
        m = g_refs[i][...] + t_ref[i][None, :, :]
        c = g_refs[i + 1][...] + t_ref[i + 1][None, :, :]
        scores = jnp.sum(m * u[:, None, :], axis=2)
        smax = jnp.max(scores, axis=1, keepdims=True)
        e = jnp.exp(scores - smax)
        p = e / jnp.sum(e, axis=1, keepdims=True)
        o = jnp.sum(p[:, :, None] * c, axis=1)
        u = o + u
    w_ref[...] = o + u


def _hops(g4, u0, tst):
    gspec = pl.BlockSpec((BT, M, D), lambda i: (i, 0, 0))
    return pl.pallas_call(
        _hops_body,
        grid=(B // BT,),
        in_specs=[
            gspec, gspec, gspec, gspec,
            pl.BlockSpec((BT, D), lambda i: (i, 0)),
            pl.BlockSpec((4, M, D), lambda i: (0, 0, 0)),
        ],
        out_specs=pl.BlockSpec((BT, D), lambda i: (i, 0)),
        out_shape=jax.ShapeDtypeStruct((B, D), jnp.float32),
    )(*g4, u0, tst)


VT = 2048
NV = (V + VT - 1) // VT


def _mm_body(e3t_ref, w_ref, o_ref):
    o_ref[...] = lax.dot_general(
        e3t_ref[...], w_ref[...],
        (((0,), (1,)), ((), ())),
        preferred_element_type=jnp.float32,
    )


def _mm(w, e3t):
    return pl.pallas_call(
        _mm_body,
        grid=(NV,),
        in_specs=[
            pl.BlockSpec((D, VT), lambda i: (0, i)),
            pl.BlockSpec((B, D), lambda i: (0, 0)),
        ],
        out_specs=pl.BlockSpec((VT, B), lambda i: (i, 0)),
        out_shape=jax.ShapeDtypeStruct((V, B), jnp.float32),
    )(e3t, w)


def kernel(x, q, E0, E1, E2, E3, T0, T1, T2, T3):
    x1d = x.astype(jnp.int32).reshape(B * M * S)
    q1d = q.astype(jnp.int32).reshape(B * S)
    g0, u0 = _sc_embed_q(x1d, q1d, E0)
    g1 = _sc_embed(x1d, E1)
    g2 = _sc_embed(x1d, E2)
    g3 = _sc_embed(x1d, E3)
    g4 = [g.reshape(B, M, D) for g in (g0, g1, g2, g3)]
    tst = jnp.stack([T0, T1, T2, T3])
    w = _hops(g4, u0, tst)
    out_t = _mm(w, E3.T)
    return out_t.T
